# Initial kernel scaffold; baseline (speedup 1.0000x reference)
import jax
import jax.numpy as jnp
from jax.experimental import pallas as pl


def kernel(seq1, seq2, adj, sparse, samp_bias1, samp_bias2, W1, b1, a1, Wb, bb):
    raise NotImplementedError("write your pallas kernel here")


# speedup vs baseline: 1.1497x; 1.1497x over previous
import functools

import jax
import jax.numpy as jnp
from jax.experimental import pallas as pl
from jax.experimental.pallas import tpu as pltpu


def _fts_kernel(s1_ref, s2_ref, w_ref, fts_ref):
    f1 = jnp.dot(s1_ref[...], w_ref[...], preferred_element_type=jnp.float32)
    f2 = jnp.dot(s2_ref[...], w_ref[...], preferred_element_type=jnp.float32)
    fts_ref[...] = jnp.concatenate([f1, f2], axis=1)


def _agg_kernel(nh, nblocks, fts_ref, ab_ref, adj_ref, h1_ref, h2_ref, hsum_ref):
    i = pl.program_id(0)

    @pl.when(i == 0)
    def _():
        hsum_ref[...] = jnp.zeros_like(hsum_ref)

    agg = jnp.dot(adj_ref[...], fts_ref[...], preferred_element_type=jnp.float32)
    b = ab_ref[0:1, :]
    a = ab_ref[1:2, :]
    z = agg + b
    h = jnp.where(z > 0, z, a * z)
    h1 = h[:, :nh]
    h2 = h[:, nh:]
    h1_ref[...] = h1
    h2_ref[...] = h2
    hsum_ref[...] += jnp.sum(h1, axis=0, keepdims=True)


def _disc_kernel(inv_n, h1_ref, h2_ref, hsum_ref, wbt_ref, sb1_ref, sb2_ref,
                 bb_ref, sc1_ref, sc2_ref):
    c = jax.nn.sigmoid(hsum_ref[...] * inv_n)
    u = jnp.dot(c, wbt_ref[...], preferred_element_type=jnp.float32)
    bbv = bb_ref[0, 0]
    sc1_ref[...] = (jnp.sum(h1_ref[...] * u, axis=1, keepdims=True)
                    + bbv + sb1_ref[...])
    sc2_ref[...] = (jnp.sum(h2_ref[...] * u, axis=1, keepdims=True)
                    + bbv + sb2_ref[...])


def kernel(seq1, seq2, adj, sparse, samp_bias1, samp_bias2, W1, b1, a1, Wb, bb):
    n = seq1.shape[1]
    nin = seq1.shape[2]
    nh = W1.shape[0]

    s1 = seq1[0]
    s2 = seq2[0]
    A = adj[0]
    wt = W1.T
    bcat = jnp.concatenate([b1, b1])
    acat = jnp.broadcast_to(a1.astype(jnp.float32), (2 * nh,))
    ab = jnp.stack([bcat, acat], axis=0)

    BR = 2000
    fts = pl.pallas_call(
        _fts_kernel,
        grid=(n // BR,),
        in_specs=[
            pl.BlockSpec((BR, nin), lambda i: (i, 0)),
            pl.BlockSpec((BR, nin), lambda i: (i, 0)),
            pl.BlockSpec((nin, nh), lambda i: (0, 0)),
        ],
        out_specs=pl.BlockSpec((BR, 2 * nh), lambda i: (i, 0)),
        out_shape=jax.ShapeDtypeStruct((n, 2 * nh), jnp.float32),
    )(s1, s2, wt)

    BM = 80
    nblocks = n // BM
    h1, h2, hsum = pl.pallas_call(
        functools.partial(_agg_kernel, nh, nblocks),
        grid=(nblocks,),
        in_specs=[
            pl.BlockSpec((n, 2 * nh), lambda i: (0, 0)),
            pl.BlockSpec((2, 2 * nh), lambda i: (0, 0)),
            pl.BlockSpec((BM, n), lambda i: (i, 0)),
        ],
        out_specs=[
            pl.BlockSpec((BM, nh), lambda i: (i, 0)),
            pl.BlockSpec((BM, nh), lambda i: (i, 0)),
            pl.BlockSpec((1, nh), lambda i: (0, 0)),
        ],
        out_shape=[
            jax.ShapeDtypeStruct((n, nh), jnp.float32),
            jax.ShapeDtypeStruct((n, nh), jnp.float32),
            jax.ShapeDtypeStruct((1, nh), jnp.float32),
        ],
    )(fts, ab, A)

    sb1 = samp_bias1.reshape(n, 1)
    sb2 = samp_bias2.reshape(n, 1)
    wbt = Wb[0].T
    bbr = bb.reshape(1, 1)
    sc1, sc2 = pl.pallas_call(
        functools.partial(_disc_kernel, 1.0 / n),
        out_shape=[jax.ShapeDtypeStruct((n, 1), jnp.float32)] * 2,
    )(h1, h2, hsum, wbt, sb1, sb2, bbr)

    logits = jnp.concatenate([sc1[:, 0], sc2[:, 0]])[None, :]
    return (logits, h1)
